# baseline (speedup 1.0000x reference)
import functools

import jax
import jax.numpy as jnp
from jax import lax
from jax.experimental import pallas as pl
from jax.experimental.pallas import tpu as pltpu
from jax.experimental.pallas import tpu_sc as plsc

NUM_ELEMENTS = 120
EMBED_DIM = 64
N_ATOMS = 100000

NC = 2
NS = 16
NW = NC * NS

PER_W = N_ATOMS // NW
CHUNK = 128
NCH = PER_W // CHUNK
TAIL = PER_W - NCH * CHUNK
PER_W_PAD = ((PER_W + 7) // 8) * 8
NBUF = 4


def _gather_body(table_hbm, idx_hbm, out_hbm, idx_v, rows_v, tail_v,
                 gsem, ssem, tsem):
    wid = lax.axis_index("s") * NC + lax.axis_index("c")
    base = wid * PER_W
    pltpu.sync_copy(idx_hbm.at[wid], idx_v)

    def mk_gather(c):
        return pltpu.make_async_copy(
            table_hbm.at[idx_v.at[pl.ds(c * CHUNK, CHUNK)]],
            rows_v.at[c % NBUF],
            gsem.at[c % NBUF],
        )

    def mk_store(c):
        return pltpu.make_async_copy(
            rows_v.at[c % NBUF],
            out_hbm.at[pl.ds(base + c * CHUNK, CHUNK)],
            ssem.at[c % NBUF],
        )

    pltpu.make_async_copy(
        table_hbm.at[idx_v.at[pl.ds(NCH * CHUNK, TAIL)]], tail_v, tsem
    ).start()
    for b in range(NBUF - 1):
        mk_gather(b).start()

    def step(j, carry):
        mk_gather(j).wait()
        mk_store(j).start()

        @pl.when(j >= 1)
        def _():
            mk_store(j - 1).wait()

        @pl.when(j + NBUF - 1 < NCH)
        def _():
            mk_gather(j + NBUF - 1).start()

        return carry

    lax.fori_loop(0, NCH, step, 0, unroll=False)

    mk_store(NCH - 1).wait()
    pltpu.make_async_copy(
        table_hbm.at[idx_v.at[pl.ds(NCH * CHUNK, TAIL)]], tail_v, tsem
    ).wait()
    pltpu.sync_copy(tail_v, out_hbm.at[pl.ds(base + NCH * CHUNK, TAIL)])


@jax.jit
def _sc_gather(table, idx_pad):
    mesh = plsc.VectorSubcoreMesh(core_axis_name="c", subcore_axis_name="s")
    f = functools.partial(
        pl.kernel,
        out_type=jax.ShapeDtypeStruct((N_ATOMS, EMBED_DIM), jnp.float32),
        mesh=mesh,
        scratch_types=[
            pltpu.VMEM((PER_W_PAD,), jnp.int32),
            pltpu.VMEM((NBUF, CHUNK, EMBED_DIM), jnp.float32),
            pltpu.VMEM((TAIL, EMBED_DIM), jnp.float32),
            pltpu.SemaphoreType.DMA((NBUF,)),
            pltpu.SemaphoreType.DMA((NBUF,)),
            pltpu.SemaphoreType.DMA,
        ],
        compiler_params=pltpu.CompilerParams(use_tc_tiling_on_sc=False),
    )(_gather_body)
    return f(table, idx_pad)


def kernel(atomic_numbers, table):
    idx = atomic_numbers.astype(jnp.int32).reshape(NW, PER_W)
    idx_pad = jnp.pad(idx, ((0, 0), (0, PER_W_PAD - PER_W)))
    return _sc_gather(table, idx_pad)


# speedup vs baseline: 1.0080x; 1.0080x over previous
import functools

import jax
import jax.numpy as jnp
from jax import lax
from jax.experimental import pallas as pl
from jax.experimental.pallas import tpu as pltpu
from jax.experimental.pallas import tpu_sc as plsc

NUM_ELEMENTS = 120
EMBED_DIM = 64
N_ATOMS = 100000

NC = 2
NS = 16
NW = NC * NS

PER_W = N_ATOMS // NW
CHUNK = 512
NCH = PER_W // CHUNK
TAIL = PER_W - NCH * CHUNK
PER_W_PAD = ((PER_W + 7) // 8) * 8
NBUF = 3


def _gather_body(table_hbm, idx_hbm, out_hbm, idx_v, rows_v, tail_v,
                 gsem, ssem, tsem):
    wid = lax.axis_index("s") * NC + lax.axis_index("c")
    base = wid * PER_W
    pltpu.sync_copy(idx_hbm.at[wid], idx_v)

    def mk_gather(c):
        return pltpu.make_async_copy(
            table_hbm.at[idx_v.at[pl.ds(c * CHUNK, CHUNK)]],
            rows_v.at[c % NBUF],
            gsem.at[c % NBUF],
        )

    def mk_store(c):
        return pltpu.make_async_copy(
            rows_v.at[c % NBUF],
            out_hbm.at[pl.ds(base + c * CHUNK, CHUNK)],
            ssem.at[c % NBUF],
        )

    pltpu.make_async_copy(
        table_hbm.at[idx_v.at[pl.ds(NCH * CHUNK, TAIL)]], tail_v, tsem
    ).start()
    for b in range(NBUF - 1):
        mk_gather(b).start()

    def step(j, carry):
        mk_gather(j).wait()
        mk_store(j).start()

        @pl.when(j >= 1)
        def _():
            mk_store(j - 1).wait()

        @pl.when(j + NBUF - 1 < NCH)
        def _():
            mk_gather(j + NBUF - 1).start()

        return carry

    lax.fori_loop(0, NCH, step, 0, unroll=False)

    mk_store(NCH - 1).wait()
    pltpu.make_async_copy(
        table_hbm.at[idx_v.at[pl.ds(NCH * CHUNK, TAIL)]], tail_v, tsem
    ).wait()
    pltpu.sync_copy(tail_v, out_hbm.at[pl.ds(base + NCH * CHUNK, TAIL)])


@jax.jit
def _sc_gather(table, idx_pad):
    mesh = plsc.VectorSubcoreMesh(core_axis_name="c", subcore_axis_name="s")
    f = functools.partial(
        pl.kernel,
        out_type=jax.ShapeDtypeStruct((N_ATOMS, EMBED_DIM), jnp.float32),
        mesh=mesh,
        scratch_types=[
            pltpu.VMEM((PER_W_PAD,), jnp.int32),
            pltpu.VMEM((NBUF, CHUNK, EMBED_DIM), jnp.float32),
            pltpu.VMEM((TAIL, EMBED_DIM), jnp.float32),
            pltpu.SemaphoreType.DMA((NBUF,)),
            pltpu.SemaphoreType.DMA((NBUF,)),
            pltpu.SemaphoreType.DMA,
        ],
        compiler_params=pltpu.CompilerParams(use_tc_tiling_on_sc=False),
    )(_gather_body)
    return f(table, idx_pad)


def kernel(atomic_numbers, table):
    idx = atomic_numbers.astype(jnp.int32).reshape(NW, PER_W)
    idx_pad = jnp.pad(idx, ((0, 0), (0, PER_W_PAD - PER_W)))
    return _sc_gather(table, idx_pad)


# speedup vs baseline: 1.6720x; 1.6587x over previous
import functools

import jax
import jax.numpy as jnp
from jax import lax
from jax.experimental import pallas as pl
from jax.experimental.pallas import tpu as pltpu
from jax.experimental.pallas import tpu_sc as plsc

NUM_ELEMENTS = 120
EMBED_DIM = 64
N_ATOMS = 100000

NC = 2
NS = 16
NW = NC * NS

PER_W = N_ATOMS // NW
CHUNK = 512
NCH = PER_W // CHUNK
TAIL = PER_W - NCH * CHUNK
PER_W_PAD = ((PER_W + 7) // 8) * 8
NBUF = 3


def _gather_body(table_hbm, idx_hbm, out_hbm, idx_v, table_sh, rows_v, tail_v,
                 gsem, ssem, tsem):
    sid = lax.axis_index("s")
    wid = sid * NC + lax.axis_index("c")
    base = wid * PER_W
    @pl.when(sid == 0)
    def _():
        pltpu.sync_copy(table_hbm, table_sh)

    pltpu.sync_copy(idx_hbm.at[wid], idx_v)
    plsc.subcore_barrier()

    def mk_gather(c):
        return pltpu.make_async_copy(
            table_sh.at[idx_v.at[pl.ds(c * CHUNK, CHUNK)]],
            rows_v.at[c % NBUF],
            gsem.at[c % NBUF],
        )

    def mk_store(c):
        return pltpu.make_async_copy(
            rows_v.at[c % NBUF],
            out_hbm.at[pl.ds(base + c * CHUNK, CHUNK)],
            ssem.at[c % NBUF],
        )

    pltpu.make_async_copy(
        table_sh.at[idx_v.at[pl.ds(NCH * CHUNK, TAIL)]], tail_v, tsem
    ).start()
    for b in range(NBUF - 1):
        mk_gather(b).start()

    def step(j, carry):
        mk_gather(j).wait()
        mk_store(j).start()

        @pl.when(j >= 1)
        def _():
            mk_store(j - 1).wait()

        @pl.when(j + NBUF - 1 < NCH)
        def _():
            mk_gather(j + NBUF - 1).start()

        return carry

    lax.fori_loop(0, NCH, step, 0, unroll=False)

    mk_store(NCH - 1).wait()
    pltpu.make_async_copy(
        table_sh.at[idx_v.at[pl.ds(NCH * CHUNK, TAIL)]], tail_v, tsem
    ).wait()
    pltpu.sync_copy(tail_v, out_hbm.at[pl.ds(base + NCH * CHUNK, TAIL)])


@jax.jit
def _sc_gather(table, idx_pad):
    mesh = plsc.VectorSubcoreMesh(core_axis_name="c", subcore_axis_name="s")
    f = functools.partial(
        pl.kernel,
        out_type=jax.ShapeDtypeStruct((N_ATOMS, EMBED_DIM), jnp.float32),
        mesh=mesh,
        scratch_types=[
            pltpu.VMEM((PER_W_PAD,), jnp.int32),
            pltpu.VMEM_SHARED((NUM_ELEMENTS, EMBED_DIM), jnp.float32),
            pltpu.VMEM((NBUF, CHUNK, EMBED_DIM), jnp.float32),
            pltpu.VMEM((TAIL, EMBED_DIM), jnp.float32),
            pltpu.SemaphoreType.DMA((NBUF,)),
            pltpu.SemaphoreType.DMA((NBUF,)),
            pltpu.SemaphoreType.DMA,
        ],
        compiler_params=pltpu.CompilerParams(use_tc_tiling_on_sc=False),
    )(_gather_body)
    return f(table, idx_pad)


def kernel(atomic_numbers, table):
    idx = atomic_numbers.astype(jnp.int32).reshape(NW, PER_W)
    idx_pad = jnp.pad(idx, ((0, 0), (0, PER_W_PAD - PER_W)))
    return _sc_gather(table, idx_pad)


# speedup vs baseline: 1.7083x; 1.0217x over previous
import functools

import jax
import jax.numpy as jnp
from jax import lax
from jax.experimental import pallas as pl
from jax.experimental.pallas import tpu as pltpu
from jax.experimental.pallas import tpu_sc as plsc

NUM_ELEMENTS = 120
EMBED_DIM = 64
N_ATOMS = 100000

NC = 2
NS = 16
NW = NC * NS

PER_W = ((N_ATOMS // NW + 7) // 8) * 8
PER_LAST = N_ATOMS - (NW - 1) * PER_W

CHUNK = 256
NCH = PER_W // CHUNK
TAIL = PER_W - NCH * CHUNK
NCH_L = PER_LAST // CHUNK
TAIL_L = PER_LAST - NCH_L * CHUNK
NBUF = 3


def _gather_body(table_hbm, idx_hbm, out_hbm, idx_v, table_sh, rows_v, tail_v,
                 gsem, ssem, tsem):
    sid = lax.axis_index("s")
    wid = sid * NC + lax.axis_index("c")
    base = wid * PER_W
    @pl.when(sid == 0)
    def _():
        pltpu.sync_copy(table_hbm, table_sh)

    def pipeline(n_idx, nch, tail):
        pltpu.sync_copy(idx_hbm.at[pl.ds(base, n_idx)],
                        idx_v.at[pl.ds(0, n_idx)])
        plsc.subcore_barrier()

        def mk_gather(c):
            return pltpu.make_async_copy(
                table_sh.at[idx_v.at[pl.ds(c * CHUNK, CHUNK)]],
                rows_v.at[c % NBUF],
                gsem.at[c % NBUF],
            )

        def mk_store(c):
            return pltpu.make_async_copy(
                rows_v.at[c % NBUF],
                out_hbm.at[pl.ds(base + c * CHUNK, CHUNK)],
                ssem.at[c % NBUF],
            )

        def mk_tail_gather():
            return pltpu.make_async_copy(
                table_sh.at[idx_v.at[pl.ds(nch * CHUNK, tail)]],
                tail_v.at[pl.ds(0, tail)],
                tsem,
            )

        mk_tail_gather().start()
        for b in range(NBUF - 1):
            mk_gather(b).start()

        def step(j, carry):
            mk_gather(j).wait()
            mk_store(j).start()

            @pl.when(j >= 1)
            def _():
                mk_store(j - 1).wait()

            @pl.when(j + NBUF - 1 < nch)
            def _():
                mk_gather(j + NBUF - 1).start()

            return carry

        lax.fori_loop(0, nch, step, 0, unroll=False)

        mk_store(nch - 1).wait()
        mk_tail_gather().wait()
        pltpu.sync_copy(tail_v.at[pl.ds(0, tail)],
                        out_hbm.at[pl.ds(base + nch * CHUNK, tail)])

    @pl.when(wid < NW - 1)
    def _():
        pipeline(PER_W, NCH, TAIL)

    @pl.when(wid == NW - 1)
    def _():
        pipeline(PER_LAST, NCH_L, TAIL_L)


@jax.jit
def _sc_gather(table, idx):
    mesh = plsc.VectorSubcoreMesh(core_axis_name="c", subcore_axis_name="s")
    f = functools.partial(
        pl.kernel,
        out_type=jax.ShapeDtypeStruct((N_ATOMS, EMBED_DIM), jnp.float32),
        mesh=mesh,
        scratch_types=[
            pltpu.VMEM((PER_W,), jnp.int32),
            pltpu.VMEM_SHARED((NUM_ELEMENTS, EMBED_DIM), jnp.float32),
            pltpu.VMEM((NBUF, CHUNK, EMBED_DIM), jnp.float32),
            pltpu.VMEM((TAIL_L, EMBED_DIM), jnp.float32),
            pltpu.SemaphoreType.DMA((NBUF,)),
            pltpu.SemaphoreType.DMA((NBUF,)),
            pltpu.SemaphoreType.DMA,
        ],
        compiler_params=pltpu.CompilerParams(use_tc_tiling_on_sc=False),
    )(_gather_body)
    return f(table, idx)


def kernel(atomic_numbers, table):
    return _sc_gather(table, atomic_numbers.astype(jnp.int32))


# speedup vs baseline: 2.2243x; 1.3021x over previous
import functools

import jax
import jax.numpy as jnp
from jax import lax
from jax.experimental import pallas as pl
from jax.experimental.pallas import tpu as pltpu
from jax.experimental.pallas import tpu_sc as plsc

NUM_ELEMENTS = 120
EMBED_DIM = 64
N_ATOMS = 100000

NC = 2
NS = 16
NW = NC * NS

PER_W = ((N_ATOMS // NW + 7) // 8) * 8
PER_LAST = N_ATOMS - (NW - 1) * PER_W

CHUNK = 256
NCH = PER_W // CHUNK
TAIL = PER_W - NCH * CHUNK
NCH_L = PER_LAST // CHUNK
TAIL_L = PER_LAST - NCH_L * CHUNK
NBUF = 3


def _gather_body(table_hbm, idx_hbm, out_hbm, idx_v, table_sh, rows_v, tail_v,
                 gsem, ssem, tsem):
    sid = lax.axis_index("s")
    wid = sid * NC + lax.axis_index("c")
    base = wid * PER_W
    @pl.when(sid == 0)
    def _():
        pltpu.sync_copy(table_hbm, table_sh)

    def pipeline(n_idx, nch, tail):
        pltpu.sync_copy(idx_hbm.at[pl.ds(base, n_idx)],
                        idx_v.at[pl.ds(0, n_idx)])
        plsc.subcore_barrier()

        def mk_gather(c):
            return pltpu.make_async_copy(
                table_sh.at[idx_v.at[pl.ds(c * CHUNK, CHUNK)]],
                rows_v.at[c % NBUF],
                gsem.at[c % NBUF],
            )

        def mk_store(c):
            return pltpu.make_async_copy(
                rows_v.at[c % NBUF],
                out_hbm.at[pl.ds(base + c * CHUNK, CHUNK)],
                ssem.at[c % NBUF],
            )

        def mk_tail_gather():
            return pltpu.make_async_copy(
                table_sh.at[idx_v.at[pl.ds(nch * CHUNK, tail)]],
                tail_v.at[pl.ds(0, tail)],
                tsem,
            )

        mk_tail_gather().start()
        for b in range(NBUF - 1):
            mk_gather(b).start()

        def step(j, carry):
            mk_gather(j).wait()
            mk_store(j).start()

            @pl.when(j >= 1)
            def _():
                mk_store(j - 1).wait()

            @pl.when(j + NBUF - 1 < nch)
            def _():
                mk_gather(j + NBUF - 1).start()

            return carry

        lax.fori_loop(0, nch, step, 0, unroll=False)

        mk_store(nch - 1).wait()
        mk_tail_gather().wait()
        pltpu.sync_copy(tail_v.at[pl.ds(0, tail)],
                        out_hbm.at[pl.ds(base + nch * CHUNK, tail)])

    @pl.when(wid < NW - 1)
    def _():
        pipeline(PER_W, NCH, TAIL)

    @pl.when(wid == NW - 1)
    def _():
        pipeline(PER_LAST, NCH_L, TAIL_L)


@jax.jit
def _sc_gather(table, idx):
    mesh = plsc.VectorSubcoreMesh(core_axis_name="c", subcore_axis_name="s")
    f = functools.partial(
        pl.kernel,
        out_type=jax.ShapeDtypeStruct((N_ATOMS, EMBED_DIM), jnp.float32),
        mesh=mesh,
        scratch_types=[
            pltpu.VMEM((PER_W,), jnp.int32),
            pltpu.VMEM_SHARED((NUM_ELEMENTS, EMBED_DIM), jnp.float32),
            pltpu.VMEM((NBUF, CHUNK, EMBED_DIM), jnp.float32),
            pltpu.VMEM((TAIL_L, EMBED_DIM), jnp.float32),
            pltpu.SemaphoreType.DMA((NBUF,)),
            pltpu.SemaphoreType.DMA((NBUF,)),
            pltpu.SemaphoreType.DMA,
        ],
        compiler_params=pltpu.CompilerParams(use_tc_tiling_on_sc=True),
    )(_gather_body)
    return f(table, idx)


def kernel(atomic_numbers, table):
    return _sc_gather(table, atomic_numbers.astype(jnp.int32))
